# baseline (speedup 1.0000x reference)
import functools

import jax
import jax.numpy as jnp
from jax import lax
from jax.experimental import pallas as pl
from jax.experimental.pallas import tpu as pltpu
from jax.experimental.pallas import tpu_sc as plsc

_B, _C, _H, _W, _K = 8, 80, 64, 64, 128
_EPS = 0.0001
_LOG2E = 1.4426950408889634
_LN2 = 0.6931471805599453

_NW = 32
_ROWS = _B * _K
_RPW = _ROWS // _NW


def _sc_gather(tag_tl, off_tl, tag_br, off_br,
               rt_tl, r0_tl, r1_tl, rt_br, r0_br, r1_br,
               o_t_tl, o_0_tl, o_1_tl, o_t_br, o_0_br, o_1_br,
               idx_v, rows_v, sem):
    wid = lax.axis_index("s") * 2 + lax.axis_index("c")
    base = wid * _RPW
    combos = (
        (tag_tl, rt_tl, o_t_tl), (off_tl, r0_tl, o_0_tl),
        (off_tl, r1_tl, o_1_tl), (tag_br, rt_br, o_t_br),
        (off_br, r0_br, o_0_br), (off_br, r1_br, o_1_br),
    )
    for tab, ridx, out in combos:
        pltpu.sync_copy(ridx.at[pl.ds(base, _RPW)], idx_v)
        pltpu.async_copy(tab.at[idx_v], rows_v, sem).wait()
        pltpu.sync_copy(rows_v, out.at[pl.ds(base, _RPW)])


def _run_sc_gather(tag_tl, off_tl, tag_br, off_br, rows):
    mesh = plsc.VectorSubcoreMesh(core_axis_name="c", subcore_axis_name="s")
    out_t = [jax.ShapeDtypeStruct((_ROWS, 128), jnp.float32)] * 6
    kern = functools.partial(
        pl.kernel, mesh=mesh, out_type=out_t,
        scratch_types=[
            pltpu.VMEM((_RPW,), jnp.int32),
            pltpu.VMEM((_RPW, 128), jnp.float32),
            pltpu.SemaphoreType.DMA,
        ],
    )(_sc_gather)
    return kern(tag_tl, off_tl, tag_br, off_br, *rows)


def _focal_elem(x, g):
    xs = x - 2.0 * (g * x)
    e = jnp.exp2(xs * _LOG2E)
    u = 1.0 + e
    l = jnp.log2(u) * (-_LN2)
    w = e / u
    return l * w * w


def _focal_part(x, gt):
    return jnp.sum(_focal_elem(x, gt)), jnp.sum(gt)


def _focal_chunked(x1_ref, g1_ref, v1_ref, x2_ref, g2_ref, v2_ref):
    zero = jnp.zeros((_H, _W), jnp.float32)

    def body(c, carry):
        a1, n1, a2, n2 = carry
        a1 = a1 + _focal_elem(x1_ref[0, c], g1_ref[0, c]) * v1_ref[0, c]
        n1 = n1 + g1_ref[0, c]
        a2 = a2 + _focal_elem(x2_ref[0, c], g2_ref[0, c]) * v2_ref[0, c]
        n2 = n2 + g2_ref[0, c]
        return a1, n1, a2, n2

    a1, n1, a2, n2 = lax.fori_loop(0, _C, body, (zero, zero, zero, zero))
    return jnp.sum(a1), jnp.sum(n1), jnp.sum(a2), jnp.sum(n2)


def _big_body(tlx, brx, gtl, gbr, vtl, vbr,
              a0, a1, a2, ga0, ga1, ga2, out_ref, acc):
    i = pl.program_id(0)

    @pl.when(i == 0)
    def _init():
        att_total = 0.0
        for a_ref, g_ref in ((a0, ga0), (a1, ga1), (a2, ga2)):
            s, npos = _focal_part(a_ref[...], g_ref[...])
            att_total += -s / npos
        acc[0] = 0.0
        acc[1] = 0.0
        acc[2] = 0.0
        acc[3] = 0.0
        acc[4] = att_total

    s_tl, np_tl, s_br, np_br = _focal_chunked(tlx, gtl, vtl, brx, gbr, vbr)
    acc[0] += s_tl
    acc[1] += np_tl
    acc[2] += s_br
    acc[3] += np_br

    @pl.when(i == _B - 1)
    def _fin():
        for t in range(5):
            out_ref[0, t] = acc[t]


def _run_big(tl_heat, br_heat, gt_tl_heat, gt_br_heat, gt_tl_valid,
             gt_br_valid, atts, interpret=False):
    big = pl.BlockSpec((1, _C, _H, _W), lambda i: (i, 0, 0, 0))

    def full(shape):
        return pl.BlockSpec(shape, lambda i: (0,) * len(shape))

    att_shapes = [(_B, 1, 16, 16), (_B, 1, 32, 32), (_B, 1, _H, _W)] * 2
    out = pl.pallas_call(
        _big_body,
        grid=(_B,),
        in_specs=[big] * 6 + [full(s) for s in att_shapes],
        out_specs=pl.BlockSpec(memory_space=pltpu.SMEM),
        out_shape=jax.ShapeDtypeStruct((1, 8), jnp.float32),
        scratch_shapes=[pltpu.SMEM((8,), jnp.float32)],
        interpret=interpret,
    )(tl_heat, br_heat, gt_tl_heat, gt_br_heat, gt_tl_valid, gt_br_valid,
      *atts)
    return out


def _lane_sel(rows_ref, ind_t):
    iota = lax.broadcasted_iota(jnp.int32, (_K, 128), 1)
    cols = []
    for b in range(_B):
        lo = ((ind_t[:, b : b + 1] & 127) == iota).astype(jnp.float32)
        rows = rows_ref[_K * b : _K * (b + 1), :]
        cols.append(jnp.sum(rows * lo, axis=1, keepdims=True))
    return jnp.transpose(jnp.concatenate(cols, axis=1), (1, 0))


def _comb_body(g_t_tl, g_0_tl, g_1_tl, g_t_br, g_0_br, g_1_br,
               indtl, indbr, maskf, gofftl0, gofftl1, goffbr0, goffbr1,
               partials, out_ref):
    ind_tl_t = jnp.transpose(indtl[...], (1, 0))
    ind_br_t = jnp.transpose(indbr[...], (1, 0))
    t0 = _lane_sel(g_t_tl, ind_tl_t)
    otl0 = _lane_sel(g_0_tl, ind_tl_t)
    otl1 = _lane_sel(g_1_tl, ind_tl_t)
    t1 = _lane_sel(g_t_br, ind_br_t)
    obr0 = _lane_sel(g_0_br, ind_br_t)
    obr1 = _lane_sel(g_1_br, ind_br_t)

    m = maskf[...]
    num = jnp.sum(m, axis=1, keepdims=True)
    mean = (t0 + t1) * 0.5
    pull = (jnp.sum((t0 - mean) ** 2 / (num + _EPS) * m)
            + jnp.sum((t1 - mean) ** 2 / (num + _EPS) * m))

    numtot = jnp.sum(m)

    def huber_sum(o, goff):
        d = o - goff[...]
        ad = jnp.abs(d)
        return jnp.sum(jnp.where(ad < 1.0, 0.5 * d * d, ad - 0.5) * m)

    off_total = (huber_sum(otl0, gofftl0) + huber_sum(otl1, gofftl1)
                 + huber_sum(obr0, goffbr0) + huber_sum(obr1, goffbr1)
                 ) / (numtot + _EPS)

    p = partials
    out_ref[0, 0] = (-p[0, 0] / p[0, 1] - p[0, 2] / p[0, 3] + p[0, 4]
                     + pull + off_total)


def _run_combine(gathered, indtl, indbr, maskf, goffs, partials,
                 interpret=False):
    vfull = lambda s: pl.BlockSpec(s, lambda: (0,) * len(s))
    out = pl.pallas_call(
        _comb_body,
        in_specs=[vfull((_ROWS, 128))] * 6
        + [vfull((_B, _K))] * 7
        + [pl.BlockSpec(memory_space=pltpu.SMEM)],
        out_specs=pl.BlockSpec(memory_space=pltpu.SMEM),
        out_shape=jax.ShapeDtypeStruct((1, 1), jnp.float32),
        interpret=interpret,
    )(*gathered, indtl, indbr, maskf, *goffs, partials)
    return out


def kernel(tl_heat, br_heat, tl_tag, br_tag, tl_off, br_off, att0, att1,
           att2, gt_tl_heat, gt_br_heat, gt_mask, gt_tl_off, gt_br_off,
           gt_tl_ind, gt_br_ind, gt_tl_valid, gt_br_valid, gt_att0,
           gt_att1, gt_att2):
    f32 = jnp.float32
    ind_tl = gt_tl_ind.astype(jnp.int32)
    ind_br = gt_br_ind.astype(jnp.int32)
    barange = jnp.arange(_B, dtype=jnp.int32)[:, None]
    h_tl = ind_tl >> 7
    h_br = ind_br >> 7
    rows = [
        (barange * 32 + h_tl).reshape(-1),
        (barange * 64 + h_tl).reshape(-1),
        (barange * 64 + 32 + h_tl).reshape(-1),
        (barange * 32 + h_br).reshape(-1),
        (barange * 64 + h_br).reshape(-1),
        (barange * 64 + 32 + h_br).reshape(-1),
    ]
    gathered = _run_sc_gather(
        tl_tag.reshape(_B * 32, 128), tl_off.reshape(_B * 64, 128),
        br_tag.reshape(_B * 32, 128), br_off.reshape(_B * 64, 128),
        rows)
    partials = _run_big(
        tl_heat, br_heat, gt_tl_heat, gt_br_heat, gt_tl_valid, gt_br_valid,
        (att0, att1, att2, gt_att0, gt_att1, gt_att2))
    out = _run_combine(
        gathered, ind_tl, ind_br, gt_mask.astype(f32),
        (gt_tl_off[:, :, 0], gt_tl_off[:, :, 1],
         gt_br_off[:, :, 0], gt_br_off[:, :, 1]),
        partials)
    return out.reshape(1)


# speedup vs baseline: 1.0120x; 1.0120x over previous
import functools

import jax
import jax.numpy as jnp
from jax import lax
from jax.experimental import pallas as pl
from jax.experimental.pallas import tpu as pltpu
from jax.experimental.pallas import tpu_sc as plsc

_B, _C, _H, _W, _K = 8, 80, 64, 64, 128
_EPS = 0.0001
_LOG2E = 1.4426950408889634
_LN2 = 0.6931471805599453

_NW = 32
_ROWS = _B * _K
_RPW = _ROWS // _NW


def _sc_gather(tag_tl, off_tl, tag_br, off_br,
               rt_tl, r0_tl, r1_tl, rt_br, r0_br, r1_br,
               lane_tl, lane_br,
               o_t_tl, o_0_tl, o_1_tl, o_t_br, o_0_br, o_1_br,
               idx_v, lanes_v, rows_v, vals_v, sem):
    wid = lax.axis_index("s") * 2 + lax.axis_index("c")
    base = wid * _RPW
    combos = (
        (tag_tl, rt_tl, lane_tl, o_t_tl), (off_tl, r0_tl, lane_tl, o_0_tl),
        (off_tl, r1_tl, lane_tl, o_1_tl), (tag_br, rt_br, lane_br, o_t_br),
        (off_br, r0_br, lane_br, o_0_br), (off_br, r1_br, lane_br, o_1_br),
    )
    for tab, ridx, lanes, out in combos:
        pltpu.sync_copy(ridx.at[pl.ds(base, _RPW)], idx_v)
        pltpu.sync_copy(lanes.at[pl.ds(base, _RPW)], lanes_v)
        pltpu.async_copy(tab.at[idx_v], rows_v, sem).wait()
        for c in range(_RPW // 16):
            rid = jnp.arange(16, dtype=jnp.int32) + 16 * c
            lid = lanes_v[pl.ds(16 * c, 16)]
            vals_v[pl.ds(16 * c, 16)] = plsc.load_gather(rows_v, [rid, lid])
        pltpu.sync_copy(vals_v, out.at[pl.ds(base, _RPW)])


def _run_sc_gather(tag_tl, off_tl, tag_br, off_br, rows, lanes):
    mesh = plsc.VectorSubcoreMesh(core_axis_name="c", subcore_axis_name="s")
    out_t = [jax.ShapeDtypeStruct((_ROWS,), jnp.float32)] * 6
    kern = functools.partial(
        pl.kernel, mesh=mesh, out_type=out_t,
        compiler_params=pltpu.CompilerParams(needs_layout_passes=False),
        scratch_types=[
            pltpu.VMEM((_RPW,), jnp.int32),
            pltpu.VMEM((_RPW,), jnp.int32),
            pltpu.VMEM((_RPW, 128), jnp.float32),
            pltpu.VMEM((_RPW,), jnp.float32),
            pltpu.SemaphoreType.DMA,
        ],
    )(_sc_gather)
    return kern(tag_tl, off_tl, tag_br, off_br, *rows, *lanes)


def _focal_elem(x, g):
    xs = x - 2.0 * (g * x)
    e = jnp.exp2(xs * _LOG2E)
    u = 1.0 + e
    l = jnp.log2(u) * (-_LN2)
    w = e / u
    return l * w * w


def _focal_part(x, gt):
    return jnp.sum(_focal_elem(x, gt)), jnp.sum(gt)


def _focal_chunked(x1_ref, g1_ref, v1_ref, x2_ref, g2_ref, v2_ref):
    zero = jnp.zeros((_H, _W), jnp.float32)

    def body(c, carry):
        a1, n1, a2, n2 = carry
        a1 = a1 + _focal_elem(x1_ref[0, c], g1_ref[0, c]) * v1_ref[0, c]
        n1 = n1 + g1_ref[0, c]
        a2 = a2 + _focal_elem(x2_ref[0, c], g2_ref[0, c]) * v2_ref[0, c]
        n2 = n2 + g2_ref[0, c]
        return a1, n1, a2, n2

    a1, n1, a2, n2 = lax.fori_loop(0, _C, body, (zero, zero, zero, zero))
    return jnp.sum(a1), jnp.sum(n1), jnp.sum(a2), jnp.sum(n2)


def _big_body(tlx, brx, gtl, gbr, vtl, vbr,
              a0, a1, a2, ga0, ga1, ga2, out_ref, acc):
    i = pl.program_id(0)

    @pl.when(i == 0)
    def _init():
        att_total = 0.0
        for a_ref, g_ref in ((a0, ga0), (a1, ga1), (a2, ga2)):
            s, npos = _focal_part(a_ref[...], g_ref[...])
            att_total += -s / npos
        acc[0] = 0.0
        acc[1] = 0.0
        acc[2] = 0.0
        acc[3] = 0.0
        acc[4] = att_total

    s_tl, np_tl, s_br, np_br = _focal_chunked(tlx, gtl, vtl, brx, gbr, vbr)
    acc[0] += s_tl
    acc[1] += np_tl
    acc[2] += s_br
    acc[3] += np_br

    @pl.when(i == _B - 1)
    def _fin():
        for t in range(5):
            out_ref[0, t] = acc[t]


def _run_big(tl_heat, br_heat, gt_tl_heat, gt_br_heat, gt_tl_valid,
             gt_br_valid, atts, interpret=False):
    big = pl.BlockSpec((1, _C, _H, _W), lambda i: (i, 0, 0, 0))

    def full(shape):
        return pl.BlockSpec(shape, lambda i: (0,) * len(shape))

    att_shapes = [(_B, 1, 16, 16), (_B, 1, 32, 32), (_B, 1, _H, _W)] * 2
    out = pl.pallas_call(
        _big_body,
        grid=(_B,),
        in_specs=[big] * 6 + [full(s) for s in att_shapes],
        out_specs=pl.BlockSpec(memory_space=pltpu.SMEM),
        out_shape=jax.ShapeDtypeStruct((1, 8), jnp.float32),
        scratch_shapes=[pltpu.SMEM((8,), jnp.float32)],
        interpret=interpret,
    )(tl_heat, br_heat, gt_tl_heat, gt_br_heat, gt_tl_valid, gt_br_valid,
      *atts)
    return out


def _comb_body(t0r, otl0r, otl1r, t1r, obr0r, obr1r,
               maskf, gofftl0, gofftl1, goffbr0, goffbr1,
               partials, out_ref):
    t0 = t0r[...]
    otl0 = otl0r[...]
    otl1 = otl1r[...]
    t1 = t1r[...]
    obr0 = obr0r[...]
    obr1 = obr1r[...]

    m = maskf[...]
    num = jnp.sum(m, axis=1, keepdims=True)
    mean = (t0 + t1) * 0.5
    pull = (jnp.sum((t0 - mean) ** 2 / (num + _EPS) * m)
            + jnp.sum((t1 - mean) ** 2 / (num + _EPS) * m))

    numtot = jnp.sum(m)

    def huber_sum(o, goff):
        d = o - goff[...]
        ad = jnp.abs(d)
        return jnp.sum(jnp.where(ad < 1.0, 0.5 * d * d, ad - 0.5) * m)

    off_total = (huber_sum(otl0, gofftl0) + huber_sum(otl1, gofftl1)
                 + huber_sum(obr0, goffbr0) + huber_sum(obr1, goffbr1)
                 ) / (numtot + _EPS)

    p = partials
    out_ref[0, 0] = (-p[0, 0] / p[0, 1] - p[0, 2] / p[0, 3] + p[0, 4]
                     + pull + off_total)


def _run_combine(gathered, maskf, goffs, partials, interpret=False):
    vfull = lambda s: pl.BlockSpec(s, lambda: (0,) * len(s))
    out = pl.pallas_call(
        _comb_body,
        in_specs=[vfull((_B, _K))] * 11
        + [pl.BlockSpec(memory_space=pltpu.SMEM)],
        out_specs=pl.BlockSpec(memory_space=pltpu.SMEM),
        out_shape=jax.ShapeDtypeStruct((1, 1), jnp.float32),
        interpret=interpret,
    )(*gathered, maskf, *goffs, partials)
    return out


def kernel(tl_heat, br_heat, tl_tag, br_tag, tl_off, br_off, att0, att1,
           att2, gt_tl_heat, gt_br_heat, gt_mask, gt_tl_off, gt_br_off,
           gt_tl_ind, gt_br_ind, gt_tl_valid, gt_br_valid, gt_att0,
           gt_att1, gt_att2):
    f32 = jnp.float32
    ind_tl = gt_tl_ind.astype(jnp.int32)
    ind_br = gt_br_ind.astype(jnp.int32)
    barange = jnp.arange(_B, dtype=jnp.int32)[:, None]
    h_tl = ind_tl >> 7
    h_br = ind_br >> 7
    rows = [
        (barange * 32 + h_tl).reshape(-1),
        (barange * 64 + h_tl).reshape(-1),
        (barange * 64 + 32 + h_tl).reshape(-1),
        (barange * 32 + h_br).reshape(-1),
        (barange * 64 + h_br).reshape(-1),
        (barange * 64 + 32 + h_br).reshape(-1),
    ]
    lanes = [(ind_tl & 127).reshape(-1), (ind_br & 127).reshape(-1)]
    gathered = _run_sc_gather(
        tl_tag.reshape(_B * 32, 128), tl_off.reshape(_B * 64, 128),
        br_tag.reshape(_B * 32, 128), br_off.reshape(_B * 64, 128),
        rows, lanes)
    partials = _run_big(
        tl_heat, br_heat, gt_tl_heat, gt_br_heat, gt_tl_valid, gt_br_valid,
        (att0, att1, att2, gt_att0, gt_att1, gt_att2))
    out = _run_combine(
        [g.reshape(_B, _K) for g in gathered], gt_mask.astype(f32),
        (gt_tl_off[:, :, 0], gt_tl_off[:, :, 1],
         gt_br_off[:, :, 0], gt_br_off[:, :, 1]),
        partials)
    return out.reshape(1)


# speedup vs baseline: 1.0233x; 1.0112x over previous
import functools

import jax
import jax.numpy as jnp
from jax import lax
from jax.experimental import pallas as pl
from jax.experimental.pallas import tpu as pltpu
from jax.experimental.pallas import tpu_sc as plsc

_B, _C, _H, _W, _K = 8, 80, 64, 64, 128
_EPS = 0.0001
_LOG2E = 1.4426950408889634
_LN2 = 0.6931471805599453

_NW = 32
_ROWS = _B * _K
_RPW = _ROWS // _NW


def _sc_gather(tag_tl, off_tl, tag_br, off_br,
               rt_tl, r0_tl, r1_tl, rt_br, r0_br, r1_br,
               lane_tl, lane_br,
               o_t_tl, o_0_tl, o_1_tl, o_t_br, o_0_br, o_1_br,
               idx_v, lanes_v, rows_v, vals_v, sem):
    wid = lax.axis_index("s") * 2 + lax.axis_index("c")
    base = wid * _RPW
    combos = (
        (tag_tl, rt_tl, lane_tl, o_t_tl), (off_tl, r0_tl, lane_tl, o_0_tl),
        (off_tl, r1_tl, lane_tl, o_1_tl), (tag_br, rt_br, lane_br, o_t_br),
        (off_br, r0_br, lane_br, o_0_br), (off_br, r1_br, lane_br, o_1_br),
    )
    for tab, ridx, lanes, out in combos:
        pltpu.sync_copy(ridx.at[pl.ds(base, _RPW)], idx_v)
        pltpu.sync_copy(lanes.at[pl.ds(base, _RPW)], lanes_v)
        pltpu.async_copy(tab.at[idx_v], rows_v, sem).wait()
        for c in range(_RPW // 16):
            rid = jnp.arange(16, dtype=jnp.int32) + 16 * c
            lid = lanes_v[pl.ds(16 * c, 16)]
            vals_v[pl.ds(16 * c, 16)] = plsc.load_gather(rows_v, [rid, lid])
        pltpu.sync_copy(vals_v, out.at[pl.ds(base, _RPW)])


def _run_sc_gather(tag_tl, off_tl, tag_br, off_br, rows, lanes):
    mesh = plsc.VectorSubcoreMesh(core_axis_name="c", subcore_axis_name="s")
    out_t = [jax.ShapeDtypeStruct((_ROWS,), jnp.float32)] * 6
    kern = functools.partial(
        pl.kernel, mesh=mesh, out_type=out_t,
        compiler_params=pltpu.CompilerParams(needs_layout_passes=False),
        scratch_types=[
            pltpu.VMEM((_RPW,), jnp.int32),
            pltpu.VMEM((_RPW,), jnp.int32),
            pltpu.VMEM((_RPW, 128), jnp.float32),
            pltpu.VMEM((_RPW,), jnp.float32),
            pltpu.SemaphoreType.DMA,
        ],
    )(_sc_gather)
    return kern(tag_tl, off_tl, tag_br, off_br, *rows, *lanes)


def _focal_elem(x, g):
    xs = x - 2.0 * (g * x)
    e = jnp.exp2(xs * _LOG2E)
    u = 1.0 + e
    l = jnp.log2(u) * (-_LN2)
    w = e / u
    return l * w * w


def _focal_part(x, gt):
    return jnp.sum(_focal_elem(x, gt)), jnp.sum(gt)


def _focal_chunked(x1_ref, g1_ref, v1_ref, x2_ref, g2_ref, v2_ref):
    zero = jnp.zeros((_H, _W), jnp.float32)

    def body(c, carry):
        a1, n1, a2, n2 = carry
        a1 = a1 + _focal_elem(x1_ref[0, c], g1_ref[0, c]) * v1_ref[0, c]
        n1 = n1 + g1_ref[0, c]
        a2 = a2 + _focal_elem(x2_ref[0, c], g2_ref[0, c]) * v2_ref[0, c]
        n2 = n2 + g2_ref[0, c]
        return a1, n1, a2, n2

    a1, n1, a2, n2 = lax.fori_loop(0, _C, body, (zero, zero, zero, zero))
    return jnp.sum(a1), jnp.sum(n1), jnp.sum(a2), jnp.sum(n2)


def _big_body(tlx, brx, gtl, gbr, vtl, vbr,
              a0, a1, a2, ga0, ga1, ga2,
              t0r, otl0r, otl1r, t1r, obr0r, obr1r,
              maskf, gofftl0, gofftl1, goffbr0, goffbr1,
              out_ref, acc):
    i = pl.program_id(0)

    @pl.when(i == 0)
    def _init():
        att_total = 0.0
        for a_ref, g_ref in ((a0, ga0), (a1, ga1), (a2, ga2)):
            s, npos = _focal_part(a_ref[...], g_ref[...])
            att_total += -s / npos

        t0 = t0r[...]
        t1 = t1r[...]
        m = maskf[...]
        num = jnp.sum(m, axis=1, keepdims=True)
        mean = (t0 + t1) * 0.5
        pull = (jnp.sum((t0 - mean) ** 2 / (num + _EPS) * m)
                + jnp.sum((t1 - mean) ** 2 / (num + _EPS) * m))

        numtot = jnp.sum(m)

        def huber_sum(o_ref, goff):
            d = o_ref[...] - goff[...]
            ad = jnp.abs(d)
            return jnp.sum(jnp.where(ad < 1.0, 0.5 * d * d, ad - 0.5) * m)

        off_total = (huber_sum(otl0r, gofftl0) + huber_sum(otl1r, gofftl1)
                     + huber_sum(obr0r, goffbr0) + huber_sum(obr1r, goffbr1)
                     ) / (numtot + _EPS)

        acc[0] = 0.0
        acc[1] = 0.0
        acc[2] = 0.0
        acc[3] = 0.0
        acc[4] = att_total + pull + off_total

    s_tl, np_tl, s_br, np_br = _focal_chunked(tlx, gtl, vtl, brx, gbr, vbr)
    acc[0] += s_tl
    acc[1] += np_tl
    acc[2] += s_br
    acc[3] += np_br

    @pl.when(i == _B - 1)
    def _fin():
        out_ref[0, 0] = -acc[0] / acc[1] - acc[2] / acc[3] + acc[4]


def _run_big(tl_heat, br_heat, gt_tl_heat, gt_br_heat, gt_tl_valid,
             gt_br_valid, atts, gathered, maskf, goffs, interpret=False):
    big = pl.BlockSpec((1, _C, _H, _W), lambda i: (i, 0, 0, 0))

    def full(shape):
        return pl.BlockSpec(shape, lambda i: (0,) * len(shape))

    att_shapes = [(_B, 1, 16, 16), (_B, 1, 32, 32), (_B, 1, _H, _W)] * 2
    out = pl.pallas_call(
        _big_body,
        grid=(_B,),
        in_specs=[big] * 6 + [full(s) for s in att_shapes]
        + [full((_B, _K))] * 11,
        out_specs=pl.BlockSpec(memory_space=pltpu.SMEM),
        out_shape=jax.ShapeDtypeStruct((1, 1), jnp.float32),
        scratch_shapes=[pltpu.SMEM((8,), jnp.float32)],
        interpret=interpret,
    )(tl_heat, br_heat, gt_tl_heat, gt_br_heat, gt_tl_valid, gt_br_valid,
      *atts, *gathered, maskf, *goffs)
    return out


def _comb_body(t0r, otl0r, otl1r, t1r, obr0r, obr1r,
               maskf, gofftl0, gofftl1, goffbr0, goffbr1,
               partials, out_ref):
    t0 = t0r[...]
    otl0 = otl0r[...]
    otl1 = otl1r[...]
    t1 = t1r[...]
    obr0 = obr0r[...]
    obr1 = obr1r[...]

    m = maskf[...]
    num = jnp.sum(m, axis=1, keepdims=True)
    mean = (t0 + t1) * 0.5
    pull = (jnp.sum((t0 - mean) ** 2 / (num + _EPS) * m)
            + jnp.sum((t1 - mean) ** 2 / (num + _EPS) * m))

    numtot = jnp.sum(m)

    def huber_sum(o, goff):
        d = o - goff[...]
        ad = jnp.abs(d)
        return jnp.sum(jnp.where(ad < 1.0, 0.5 * d * d, ad - 0.5) * m)

    off_total = (huber_sum(otl0, gofftl0) + huber_sum(otl1, gofftl1)
                 + huber_sum(obr0, goffbr0) + huber_sum(obr1, goffbr1)
                 ) / (numtot + _EPS)

    p = partials
    out_ref[0, 0] = (-p[0, 0] / p[0, 1] - p[0, 2] / p[0, 3] + p[0, 4]
                     + pull + off_total)


def _run_combine(gathered, maskf, goffs, partials, interpret=False):
    vfull = lambda s: pl.BlockSpec(s, lambda: (0,) * len(s))
    out = pl.pallas_call(
        _comb_body,
        in_specs=[vfull((_B, _K))] * 11
        + [pl.BlockSpec(memory_space=pltpu.SMEM)],
        out_specs=pl.BlockSpec(memory_space=pltpu.SMEM),
        out_shape=jax.ShapeDtypeStruct((1, 1), jnp.float32),
        interpret=interpret,
    )(*gathered, maskf, *goffs, partials)
    return out


def kernel(tl_heat, br_heat, tl_tag, br_tag, tl_off, br_off, att0, att1,
           att2, gt_tl_heat, gt_br_heat, gt_mask, gt_tl_off, gt_br_off,
           gt_tl_ind, gt_br_ind, gt_tl_valid, gt_br_valid, gt_att0,
           gt_att1, gt_att2):
    f32 = jnp.float32
    ind_tl = gt_tl_ind.astype(jnp.int32)
    ind_br = gt_br_ind.astype(jnp.int32)
    barange = jnp.arange(_B, dtype=jnp.int32)[:, None]
    h_tl = ind_tl >> 7
    h_br = ind_br >> 7
    rows = [
        (barange * 32 + h_tl).reshape(-1),
        (barange * 64 + h_tl).reshape(-1),
        (barange * 64 + 32 + h_tl).reshape(-1),
        (barange * 32 + h_br).reshape(-1),
        (barange * 64 + h_br).reshape(-1),
        (barange * 64 + 32 + h_br).reshape(-1),
    ]
    lanes = [(ind_tl & 127).reshape(-1), (ind_br & 127).reshape(-1)]
    gathered = _run_sc_gather(
        tl_tag.reshape(_B * 32, 128), tl_off.reshape(_B * 64, 128),
        br_tag.reshape(_B * 32, 128), br_off.reshape(_B * 64, 128),
        rows, lanes)
    out = _run_big(
        tl_heat, br_heat, gt_tl_heat, gt_br_heat, gt_tl_valid, gt_br_valid,
        (att0, att1, att2, gt_att0, gt_att1, gt_att2),
        [g.reshape(_B, _K) for g in gathered], gt_mask.astype(f32),
        (gt_tl_off[:, :, 0], gt_tl_off[:, :, 1],
         gt_br_off[:, :, 0], gt_br_off[:, :, 1]))
    return out.reshape(1)
